# baseline (speedup 1.0000x reference)
import functools

import jax
import jax.numpy as jnp
from jax import lax
from jax.experimental import pallas as pl
from jax.experimental.pallas import tpu as pltpu
from jax.experimental.pallas import tpu_sc as plsc

VOCAB = 1_000_000
DIM = 64
NB = 4096
NS = 200
BATCH = NB * NS

NUM_CORES = 2
NUM_SUBCORES = 16
NUM_WORKERS = NUM_CORES * NUM_SUBCORES
ROWS_PER_W = NB // NUM_WORKERS
PER_WORKER = ROWS_PER_W * NS

PAIRS = VOCAB // 2
FULL_BLOCKS = VOCAB // 128
TAIL_COLS = VOCAB - FULL_BLOCKS * 128
TAIL_WORKER = FULL_BLOCKS % NUM_WORKERS

_COMPILER_PARAMS = pltpu.CompilerParams(
    use_tc_tiling_on_sc=True, needs_layout_passes=False
)


def _fmt_body(tab_t, scratch, vbuf, tbuf, tail_vbuf, *sems):
    rsems, wsems = sems[:2], sems[2:]
    wid = lax.axis_index("s") * NUM_CORES + lax.axis_index("c")
    base_blocks = FULL_BLOCKS // NUM_WORKERS
    extra = FULL_BLOCKS - base_blocks * NUM_WORKERS
    nw = base_blocks + (wid < extra).astype(jnp.int32)
    lanes = lax.iota(jnp.int32, 16)
    dvec = [16 * (q % 4) + lanes for q in range(8)]

    def read(i, c):
        return pltpu.make_async_copy(
            tab_t.at[:, pl.ds((wid + NUM_WORKERS * i) * 128, 128)],
            vbuf.at[c],
            rsems[c],
        )

    def write(i, c):
        return pltpu.make_async_copy(
            tbuf.at[c],
            scratch.at[pl.ds((wid + NUM_WORKERS * i) * 64, 64)],
            wsems[c],
        )

    def transpose(src, c, nk):
        def kbody(k4, _):
            for u in range(4):
                k = k4 * 4 + u
                cols = [
                    jnp.broadcast_to(2 * k, (16,)).astype(jnp.int32),
                    jnp.broadcast_to(2 * k + 1, (16,)).astype(jnp.int32),
                ]
                for q in range(8):
                    val = plsc.load_gather(src, [dvec[q], cols[q // 4]])
                    tbuf[c, k, pl.ds(16 * q, 16)] = val
            return 0

        lax.fori_loop(0, nk // 4, kbody, 0)

    read(0, 0).start()

    def turn(g, _):
        for c in range(2):
            i = 2 * g + c

            @pl.when(i < nw)
            def _():
                @pl.when(i + 1 < nw)
                def _():
                    read(i + 1, 1 - c).start()

                read(i, c).wait()

                @pl.when(i >= 2)
                def _():
                    write(0, c).wait()

                transpose(vbuf.at[c], c, 64)
                write(i, c).start()
        return 0

    lax.fori_loop(0, 123, turn, 0)
    for c in range(2):
        write(0, c).wait()

    @pl.when(wid == TAIL_WORKER)
    def _():
        pltpu.sync_copy(
            tab_t.at[:, pl.ds(FULL_BLOCKS * 128, TAIL_COLS)],
            tail_vbuf,
        )
        transpose(tail_vbuf, 0, TAIL_COLS // 2)
        pltpu.sync_copy(
            tbuf.at[0].at[pl.ds(0, TAIL_COLS // 2)],
            scratch.at[pl.ds(FULL_BLOCKS * 64, TAIL_COLS // 2)],
        )


def _emb_body(idx_hbm, table_hbm, out_hbm, idx_v, jbuf, pbuf, gbuf, obuf, *sems):
    gsems, osems = sems[:2], sems[2:]
    wid = lax.axis_index("s") * NUM_CORES + lax.axis_index("c")
    base = wid * PER_WORKER
    row0 = wid * ROWS_PER_W
    pltpu.sync_copy(idx_hbm.at[pl.ds(base, PER_WORKER)], idx_v)

    lanes = lax.iota(jnp.int32, 16)

    def build_idx(s, b):
        for q in range(8):
            pos = (q * 16 + lanes) * NS + s
            v = plsc.load_gather(idx_v, [pos])
            jbuf[b, pl.ds(q * 16, 16)] = lax.shift_right_logical(v, 1)
            pbuf[b, pl.ds(q * 16, 16)] = lax.bitwise_and(v, 1) * DIM

    def gather(b):
        return pltpu.make_async_copy(
            table_hbm.at[jbuf.at[b]], gbuf.at[b], gsems[b]
        )

    def outcopy(s, b):
        return pltpu.make_async_copy(
            obuf.at[b],
            out_hbm.at[s, :, pl.ds(row0, ROWS_PER_W)],
            osems[b],
        )

    def pack(b):
        rvs = [q * 16 + lanes for q in range(8)]
        pbase = [pbuf[b, pl.ds(q * 16, 16)] for q in range(8)]

        def dbody(d8, _):
            for u in range(8):
                d = d8 * 8 + u
                for q in range(8):
                    val = plsc.load_gather(gbuf.at[b], [rvs[q], pbase[q] + d])
                    obuf[b, d, pl.ds(q * 16, 16)] = val
            return 0

        lax.fori_loop(0, DIM // 8, dbody, 0)

    build_idx(0, 0)
    gather(0).start()

    def turn(g, _):
        for b in range(2):
            s = 2 * g + b

            @pl.when(s + 1 < NS)
            def _():
                build_idx(s + 1, 1 - b)
                gather(1 - b).start()

            gather(b).wait()

            @pl.when(s >= 2)
            def _():
                outcopy(s - 2, b).wait()

            pack(b)
            outcopy(s, b).start()
        return 0

    lax.fori_loop(0, NS // 2, turn, 0)
    outcopy(NS - 2, 0).wait()
    outcopy(NS - 1, 1).wait()


@jax.jit
def _embed_fused(idx_flat, table_t):
    mesh = plsc.VectorSubcoreMesh(core_axis_name="c", subcore_axis_name="s")
    fmt = functools.partial(
        pl.kernel,
        mesh=mesh,
        out_type=jax.ShapeDtypeStruct((PAIRS, 128), jnp.float32),
        scratch_types=[
            pltpu.VMEM((2, DIM, 128), jnp.float32),
            pltpu.VMEM((2, DIM, 128), jnp.float32),
            pltpu.VMEM((DIM, TAIL_COLS), jnp.float32),
        ]
        + [pltpu.SemaphoreType.DMA] * 4,
        compiler_params=_COMPILER_PARAMS,
    )(_fmt_body)
    scratch = fmt(table_t)

    emb = functools.partial(
        pl.kernel,
        mesh=mesh,
        out_type=jax.ShapeDtypeStruct((NS, DIM, NB), jnp.float32),
        scratch_types=[
            pltpu.VMEM((PER_WORKER,), jnp.int32),
            pltpu.VMEM((2, ROWS_PER_W), jnp.int32),
            pltpu.VMEM((2, ROWS_PER_W), jnp.int32),
            pltpu.VMEM((2, ROWS_PER_W, 128), jnp.float32),
            pltpu.VMEM((2, DIM, ROWS_PER_W), jnp.float32),
        ]
        + [pltpu.SemaphoreType.DMA] * 4,
        compiler_params=_COMPILER_PARAMS,
    )(_emb_body)
    return emb(idx_flat, scratch)


def kernel(token_ids, embeddings):
    b, s = token_ids.shape
    idx_flat = token_ids.reshape(b * s).astype(jnp.int32)
    table_t = jnp.transpose(embeddings)
    out = _embed_fused(idx_flat, table_t)
    return jnp.transpose(out, (2, 0, 1))


# speedup vs baseline: 2.2200x; 2.2200x over previous
import functools

import jax
import jax.numpy as jnp
from jax import lax
from jax.experimental import pallas as pl
from jax.experimental.pallas import tpu as pltpu
from jax.experimental.pallas import tpu_sc as plsc

VOCAB = 1_000_000
DIM = 64
BATCH = 4096 * 200

NUM_CORES = 2
NUM_SUBCORES = 16
NUM_WORKERS = NUM_CORES * NUM_SUBCORES
PER_WORKER = BATCH // NUM_WORKERS
NBUF = 4
CHUNK = 400
N_CHUNKS = PER_WORKER // CHUNK
N_OUTER = N_CHUNKS // NBUF


def _emb_body(idx_hbm, table_hbm, out_hbm, idx_v, rows_v, *sems):
    gsems, osems = sems[:NBUF], sems[NBUF:]
    wid = lax.axis_index("s") * NUM_CORES + lax.axis_index("c")
    base = wid * PER_WORKER
    pltpu.sync_copy(idx_hbm.at[pl.ds(base, PER_WORKER)], idx_v)

    def gather(i, b):
        return pltpu.make_async_copy(
            table_hbm.at[idx_v.at[pl.ds(i * CHUNK, CHUNK)]],
            rows_v.at[b],
            gsems[b],
        )

    def outcopy(i, b):
        return pltpu.make_async_copy(
            rows_v.at[b],
            out_hbm.at[pl.ds(base + i * CHUNK, CHUNK)],
            osems[b],
        )

    for b in range(NBUF):
        gather(b, b).start()

    def turn(g, _):
        for b in range(NBUF):
            i = g * NBUF + b
            gather(i, b).wait()
            outcopy(i, b).start()
            outcopy(i, b).wait()
            gather(i + NBUF, b).start()
        return 0

    lax.fori_loop(0, N_OUTER - 1, turn, 0)

    for b in range(NBUF):
        i = (N_OUTER - 1) * NBUF + b
        gather(i, b).wait()
        outcopy(i, b).start()
        outcopy(i, b).wait()


@jax.jit
def _embed_flat(idx_flat, table):
    mesh = plsc.VectorSubcoreMesh(core_axis_name="c", subcore_axis_name="s")
    f = functools.partial(
        pl.kernel,
        mesh=mesh,
        out_type=jax.ShapeDtypeStruct((BATCH, DIM), jnp.float32),
        scratch_types=[
            pltpu.VMEM((PER_WORKER,), jnp.int32),
            pltpu.VMEM((NBUF, CHUNK, DIM), jnp.float32),
        ]
        + [pltpu.SemaphoreType.DMA] * (2 * NBUF),
        compiler_params=pltpu.CompilerParams(use_tc_tiling_on_sc=False),
    )(_emb_body)
    return f(idx_flat, table)


def kernel(token_ids, embeddings):
    b, s = token_ids.shape
    idx_flat = token_ids.reshape(b * s).astype(jnp.int32)
    out = _embed_flat(idx_flat, embeddings)
    return out.reshape(b, s, DIM)
